# baseline (speedup 1.0000x reference)
import functools

import jax
import jax.numpy as jnp
from jax import lax
from jax.experimental import pallas as pl
from jax.experimental.pallas import tpu as pltpu
from jax.experimental.pallas import tpu_sc as plsc

N = 10000
E = 160000
D = 256
H = 128
L = 16
NTILES = 16
CHUNK = 128
NCHUNK = 80
NPASS = 2
NP2 = NCHUNK // NPASS
EPT = NCHUNK * CHUNK
EPAD = EPT * NTILES
NPAD = 10240
RPT = NPAD // NTILES


def _sc_body(xv, gidxp, dstp, wp, out, agg_sh, rows0, rows1, gtile, dtile,
             wtile, sem0, sem1, ssem0, ssem1, semg):
    c = lax.axis_index("c")
    s = lax.axis_index("s")

    pltpu.async_copy(gidxp.at[c, s, 0], gtile, semg)
    pltpu.async_copy(dstp.at[s, 0], dtile, semg)
    pltpu.async_copy(wp.at[s, 0], wtile, semg)

    zero = jnp.zeros((L,), jnp.float32)

    @plsc.parallel_loop(0, CHUNK)
    def zrow(i):
        for k in range(H // L):
            rows0[i, pl.ds(k * L, L)] = zero
    rbase = s * RPT
    for r in range(RPT // CHUNK):
        pltpu.sync_copy(rows0, agg_sh.at[pl.ds(rbase + r * CHUNK, CHUNK)])
    plsc.subcore_barrier()

    def scale(ci, rows):
        @plsc.parallel_loop(0, CHUNK // L)
        def sg(g):
            w16 = wtile[ci, pl.ds(g * L, L)]
            for j in range(L):
                e = g * L + j
                wv = jnp.full((L,), w16[j], jnp.float32)
                for k in range(H // L):
                    rows[e, pl.ds(k * L, L)] = rows[e, pl.ds(k * L, L)] * wv

    for p in range(NPASS):
        if p == 0:
            pltpu.make_async_copy(gidxp.at[c, s, 0], gtile, semg).wait()
            pltpu.make_async_copy(dstp.at[s, 0], dtile, semg).wait()
            pltpu.make_async_copy(wp.at[s, 0], wtile, semg).wait()
        else:
            pltpu.sync_copy(gidxp.at[c, s, p], gtile)
            pltpu.sync_copy(dstp.at[s, p], dtile)
            pltpu.sync_copy(wp.at[s, p], wtile)

        pltpu.async_copy(xv.at[gtile.at[0]], rows0, sem0)

        def body(j, carry):
            ci0 = 2 * j
            ci1 = 2 * j + 1
            pltpu.async_copy(xv.at[gtile.at[ci1]], rows1, sem1)
            pltpu.make_async_copy(xv.at[gtile.at[ci0]], rows0, sem0).wait()
            scale(ci0, rows0)
            pltpu.sync_copy(rows0, agg_sh.at[dtile.at[ci0]], add=True)

            @pl.when(ci0 + 2 < NP2)
            def _():
                pltpu.async_copy(xv.at[gtile.at[ci0 + 2]], rows0, sem0)

            pltpu.make_async_copy(xv.at[gtile.at[ci1]], rows1, sem1).wait()
            scale(ci1, rows1)
            pltpu.sync_copy(rows1, agg_sh.at[dtile.at[ci1]], add=True)
            return carry

        lax.fori_loop(0, NP2 // 2, body, 0)
    plsc.subcore_barrier()

    pltpu.sync_copy(agg_sh.at[pl.ds(rbase, RPT)], out.at[c, pl.ds(rbase, RPT)])


_sc_call = pl.kernel(
    _sc_body,
    mesh=plsc.VectorSubcoreMesh(core_axis_name="c", subcore_axis_name="s"),
    out_type=jax.ShapeDtypeStruct((2, NPAD, H), jnp.float32),
    scratch_types=[
        pltpu.VMEM_SHARED((NPAD, H), jnp.float32),
        pltpu.VMEM((CHUNK, H), jnp.float32),
        pltpu.VMEM((CHUNK, H), jnp.float32),
        pltpu.VMEM((NP2, CHUNK), jnp.int32),
        pltpu.VMEM((NP2, CHUNK), jnp.int32),
        pltpu.VMEM((NP2, CHUNK), jnp.float32),
        pltpu.SemaphoreType.DMA,
        pltpu.SemaphoreType.DMA,
        pltpu.SemaphoreType.DMA,
        pltpu.SemaphoreType.DMA,
        pltpu.SemaphoreType.DMA,
    ],
)


BLK = 400


def _tc_body(x_ref, a0_ref, a1_ref, w1_ref, b1_ref, w2_ref, b2_ref, o_ref,
             h_ref):
    h_ref[:, :H] = x_ref[:, :H] + a0_ref[0]
    h_ref[:, H:] = x_ref[:, H:] + a1_ref[0]
    h1 = jnp.maximum(
        jnp.dot(h_ref[...].astype(jnp.bfloat16),
                w1_ref[...].astype(jnp.bfloat16),
                preferred_element_type=jnp.float32)
        + b1_ref[...], 0.0)
    o_ref[...] = (
        jnp.dot(h1.astype(jnp.bfloat16), w2_ref[...].astype(jnp.bfloat16),
                preferred_element_type=jnp.float32)
        + b2_ref[...])


_tc_call = pl.pallas_call(
    _tc_body,
    grid=(N // BLK,),
    in_specs=[
        pl.BlockSpec((BLK, D), lambda i: (i, 0)),
        pl.BlockSpec((1, BLK, H), lambda i: (0, i, 0)),
        pl.BlockSpec((1, BLK, H), lambda i: (1, i, 0)),
        pl.BlockSpec((D, D), lambda i: (0, 0)),
        pl.BlockSpec((1, D), lambda i: (0, 0)),
        pl.BlockSpec((D, D), lambda i: (0, 0)),
        pl.BlockSpec((1, D), lambda i: (0, 0)),
    ],
    out_specs=pl.BlockSpec((BLK, D), lambda i: (i, 0)),
    out_shape=jax.ShapeDtypeStruct((N, D), jnp.float32),
    scratch_shapes=[pltpu.VMEM((BLK, D), jnp.float32)],
)


@jax.jit
def kernel(x, edge_index, edge_weight, W1, b1, W2, b2):
    xv = x.reshape(2 * N, H)
    pad = EPAD - E
    src = jnp.concatenate([edge_index[0], jnp.zeros((pad,), jnp.int32)])
    dst = jnp.concatenate([edge_index[1], jnp.zeros((pad,), jnp.int32)])
    w = jnp.concatenate([edge_weight, jnp.zeros((pad,), jnp.float32)])
    gidx = jnp.stack([2 * src, 2 * src + 1])
    agg2 = _sc_call(xv, gidx.reshape(2, NTILES, NPASS, NP2, CHUNK),
                    dst.reshape(NTILES, NPASS, NP2, CHUNK),
                    w.reshape(NTILES, NPASS, NP2, CHUNK))
    return _tc_call(x, agg2, agg2, W1, b1.reshape(1, D), W2, b2.reshape(1, D))


# speedup vs baseline: 1.0349x; 1.0349x over previous
import functools

import jax
import jax.numpy as jnp
from jax import lax
from jax.experimental import pallas as pl
from jax.experimental.pallas import tpu as pltpu
from jax.experimental.pallas import tpu_sc as plsc

N = 10000
E = 160000
D = 256
H = 128
L = 16
NTILES = 16
CHUNK = 128
NCHUNK = 80
NPASS = 2
NP2 = NCHUNK // NPASS
EPT = NCHUNK * CHUNK
EPAD = EPT * NTILES
NPAD = 10240
RPT = NPAD // NTILES


def _sc_body(xv, gidxp, dstp, wp, out, agg_sh, rows0, rows1, gtile, dtile,
             wtile, sem0, sem1, semg):
    c = lax.axis_index("c")
    s = lax.axis_index("s")

    pltpu.async_copy(gidxp.at[c, s, 0], gtile, semg)
    pltpu.async_copy(dstp.at[s, 0], dtile, semg)
    pltpu.async_copy(wp.at[s, 0], wtile, semg)

    zero = jnp.zeros((L,), jnp.float32)

    @plsc.parallel_loop(0, CHUNK)
    def zrow(i):
        for k in range(H // L):
            rows0[i, pl.ds(k * L, L)] = zero
    rbase = s * RPT
    for r in range(RPT // CHUNK):
        pltpu.sync_copy(rows0, agg_sh.at[pl.ds(rbase + r * CHUNK, CHUNK)])
    plsc.subcore_barrier()

    def scale(ci, rows):
        @plsc.parallel_loop(0, CHUNK // L)
        def sg(g):
            w16 = wtile[ci, pl.ds(g * L, L)]
            for j in range(L):
                e = g * L + j
                wv = jnp.full((L,), w16[j], jnp.float32)
                for k in range(H // L):
                    rows[e, pl.ds(k * L, L)] = rows[e, pl.ds(k * L, L)] * wv

    for p in range(NPASS):
        if p == 0:
            pltpu.make_async_copy(gidxp.at[c, s, 0], gtile, semg).wait()
            pltpu.make_async_copy(dstp.at[s, 0], dtile, semg).wait()
            pltpu.make_async_copy(wp.at[s, 0], wtile, semg).wait()
        else:
            pltpu.sync_copy(gidxp.at[c, s, p], gtile)
            pltpu.sync_copy(dstp.at[s, p], dtile)
            pltpu.sync_copy(wp.at[s, p], wtile)

        pltpu.async_copy(xv.at[gtile.at[0]], rows0, sem0)

        def body(j, carry):
            ci0 = 2 * j
            ci1 = 2 * j + 1
            pltpu.async_copy(xv.at[gtile.at[ci1]], rows1, sem1)
            pltpu.make_async_copy(xv.at[gtile.at[ci0]], rows0, sem0).wait()
            scale(ci0, rows0)
            pltpu.sync_copy(rows0, agg_sh.at[dtile.at[ci0]], add=True)

            @pl.when(ci0 + 2 < NP2)
            def _():
                pltpu.async_copy(xv.at[gtile.at[ci0 + 2]], rows0, sem0)

            pltpu.make_async_copy(xv.at[gtile.at[ci1]], rows1, sem1).wait()
            scale(ci1, rows1)
            pltpu.sync_copy(rows1, agg_sh.at[dtile.at[ci1]], add=True)
            return carry

        lax.fori_loop(0, NP2 // 2, body, 0)
    plsc.subcore_barrier()

    pltpu.sync_copy(agg_sh.at[pl.ds(rbase, RPT)], out.at[c, pl.ds(rbase, RPT)])


_sc_call = pl.kernel(
    _sc_body,
    mesh=plsc.VectorSubcoreMesh(core_axis_name="c", subcore_axis_name="s"),
    out_type=jax.ShapeDtypeStruct((2, NPAD, H), jnp.float32),
    scratch_types=[
        pltpu.VMEM_SHARED((NPAD, H), jnp.float32),
        pltpu.VMEM((CHUNK, H), jnp.float32),
        pltpu.VMEM((CHUNK, H), jnp.float32),
        pltpu.VMEM((NP2, CHUNK), jnp.int32),
        pltpu.VMEM((NP2, CHUNK), jnp.int32),
        pltpu.VMEM((NP2, CHUNK), jnp.float32),
        pltpu.SemaphoreType.DMA,
        pltpu.SemaphoreType.DMA,
        pltpu.SemaphoreType.DMA,
    ],
)


BLK = 400


def _tc_body(x_ref, a0_ref, a1_ref, w1_ref, b1_ref, w2_ref, b2_ref, o_ref,
             h_ref):
    h_ref[:, :H] = x_ref[:, :H] + a0_ref[0]
    h_ref[:, H:] = x_ref[:, H:] + a1_ref[0]
    h1 = jnp.maximum(
        jnp.dot(h_ref[...], w1_ref[...], preferred_element_type=jnp.float32)
        + b1_ref[...], 0.0)
    o_ref[...] = (
        jnp.dot(h1, w2_ref[...], preferred_element_type=jnp.float32)
        + b2_ref[...])


_tc_call = pl.pallas_call(
    _tc_body,
    grid=(N // BLK,),
    in_specs=[
        pl.BlockSpec((BLK, D), lambda i: (i, 0)),
        pl.BlockSpec((1, BLK, H), lambda i: (0, i, 0)),
        pl.BlockSpec((1, BLK, H), lambda i: (1, i, 0)),
        pl.BlockSpec((D, D), lambda i: (0, 0)),
        pl.BlockSpec((1, D), lambda i: (0, 0)),
        pl.BlockSpec((D, D), lambda i: (0, 0)),
        pl.BlockSpec((1, D), lambda i: (0, 0)),
    ],
    out_specs=pl.BlockSpec((BLK, D), lambda i: (i, 0)),
    out_shape=jax.ShapeDtypeStruct((N, D), jnp.float32),
    scratch_shapes=[pltpu.VMEM((BLK, D), jnp.float32)],
)


@jax.jit
def kernel(x, edge_index, edge_weight, W1, b1, W2, b2):
    xv = x.reshape(2 * N, H)
    pad = EPAD - E
    src = jnp.concatenate([edge_index[0], jnp.zeros((pad,), jnp.int32)])
    dst = jnp.concatenate([edge_index[1], jnp.zeros((pad,), jnp.int32)])
    w = jnp.concatenate([edge_weight, jnp.zeros((pad,), jnp.float32)])
    gidx = jnp.stack([2 * src, 2 * src + 1])
    agg2 = _sc_call(xv, gidx.reshape(2, NTILES, NPASS, NP2, CHUNK),
                    dst.reshape(NTILES, NPASS, NP2, CHUNK),
                    w.reshape(NTILES, NPASS, NP2, CHUNK))
    return _tc_call(x, agg2, agg2, W1, b1.reshape(1, D), W2, b2.reshape(1, D))
